# Initial kernel scaffold; baseline (speedup 1.0000x reference)
import jax
import jax.numpy as jnp
from jax.experimental import pallas as pl


def kernel(boxes, scores):
    raise NotImplementedError("write your pallas kernel here")


# speedup vs baseline: 23.5174x; 23.5174x over previous
import functools

import jax
import jax.numpy as jnp
from jax.experimental import pallas as pl
from jax.experimental.pallas import tpu as pltpu

CONF_THRES = 0.25
IOU_THRES = 0.45
MAX_DET = 300
N_BOXES = 20000
ROWS = 160
LANES = 128
PAD_N = ROWS * LANES


def _nms_body(cx_ref, cy_ref, w_ref, h_ref, s_ref, out_ref,
              x1_ref, y1_ref, x2_ref, y2_ref, area_ref, act_ref, idx_ref):
    cx = cx_ref[:] * 640.0
    cy = cy_ref[:] * 640.0
    w = w_ref[:] * 100.0 + 2.0
    h = h_ref[:] * 100.0 + 2.0
    x1 = cx - w * 0.5
    y1 = cy - h * 0.5
    x2 = cx + w * 0.5
    y2 = cy + h * 0.5
    x1_ref[:] = x1
    y1_ref[:] = y1
    x2_ref[:] = x2
    y2_ref[:] = y2
    area_ref[:] = (x2 - x1) * (y2 - y1)
    s = s_ref[:]
    act_ref[:] = jnp.where(s > CONF_THRES, s, -1.0)
    ridx = jax.lax.broadcasted_iota(jnp.int32, (ROWS, LANES), 0)
    lidx = jax.lax.broadcasted_iota(jnp.int32, (ROWS, LANES), 1)
    idx_ref[:] = ridx * LANES + lidx

    lane = jax.lax.broadcasted_iota(jnp.int32, (1, LANES), 1)

    def body(i, _):
        act = act_ref[:]
        idx = idx_ref[:]
        m = jnp.max(act)
        j = jnp.min(jnp.where(act == m, idx, jnp.int32(2**30)))
        v = m > 0.0
        r = j // LANES
        c = j - r * LANES
        cmask = lane == c
        neg = jnp.float32(-3e38)
        bx1 = jnp.max(jnp.where(cmask, x1_ref[pl.ds(r, 1), :], neg))
        by1 = jnp.max(jnp.where(cmask, y1_ref[pl.ds(r, 1), :], neg))
        bx2 = jnp.max(jnp.where(cmask, x2_ref[pl.ds(r, 1), :], neg))
        by2 = jnp.max(jnp.where(cmask, y2_ref[pl.ds(r, 1), :], neg))
        xx1 = jnp.maximum(bx1, x1_ref[:])
        yy1 = jnp.maximum(by1, y1_ref[:])
        xx2 = jnp.minimum(bx2, x2_ref[:])
        yy2 = jnp.minimum(by2, y2_ref[:])
        inter = jnp.maximum(xx2 - xx1, 0.0) * jnp.maximum(yy2 - yy1, 0.0)
        a1 = (bx2 - bx1) * (by2 - by1)
        iou = inter / (a1 + area_ref[:] - inter + 1e-7)
        sup = ((iou > IOU_THRES) & v) | (idx == j)
        act_ref[:] = jnp.where(sup, -1.0, act)
        vf = jnp.where(v, 1.0, 0.0)
        row = jnp.where(lane == 0, bx1 * vf,
              jnp.where(lane == 1, by1 * vf,
              jnp.where(lane == 2, bx2 * vf,
              jnp.where(lane == 3, by2 * vf,
              jnp.where(lane == 4, m * vf, 0.0)))))
        out_ref[pl.ds(i, 1), :] = row
        return 0

    jax.lax.fori_loop(0, MAX_DET, body, 0)


@jax.jit
def kernel(boxes, scores):
    pad = PAD_N - N_BOXES
    bp = jnp.pad(boxes, ((0, pad), (0, 0)))
    sp = jnp.pad(scores, (0, pad))
    cx = bp[:, 0].reshape(ROWS, LANES)
    cy = bp[:, 1].reshape(ROWS, LANES)
    w = bp[:, 2].reshape(ROWS, LANES)
    h = bp[:, 3].reshape(ROWS, LANES)
    sp = sp.reshape(ROWS, LANES)

    f32 = jnp.float32
    out = pl.pallas_call(
        _nms_body,
        out_shape=jax.ShapeDtypeStruct((MAX_DET, LANES), f32),
        scratch_shapes=[
            pltpu.VMEM((ROWS, LANES), f32),
            pltpu.VMEM((ROWS, LANES), f32),
            pltpu.VMEM((ROWS, LANES), f32),
            pltpu.VMEM((ROWS, LANES), f32),
            pltpu.VMEM((ROWS, LANES), f32),
            pltpu.VMEM((ROWS, LANES), f32),
            pltpu.VMEM((ROWS, LANES), jnp.int32),
        ],
    )(cx, cy, w, h, sp)
    return out[:, :5]
